# baseline (speedup 1.0000x reference)
import functools
import jax
import jax.numpy as jnp
from jax import lax
from jax.experimental import pallas as pl
from jax.experimental.pallas import tpu as pltpu
from jax.experimental.pallas import tpu_sc as plsc

HIDDEN = 1024
NC, NS = 2, 16
NW = NC * NS
CHUNK_ROWS = 16
CHUNK = CHUNK_ROWS * HIDDEN
UNROLL = 16


def kernel(x, table):
    b, n, h = x.shape
    rows = b * n
    ppw = n // NW
    npc = ppw // CHUNK_ROWS
    nstep = npc * b

    x2 = x.reshape(rows, h)

    mesh = plsc.VectorSubcoreMesh(
        core_axis_name="c", subcore_axis_name="s",
        num_cores=NC, num_subcores=NS)

    @functools.partial(
        pl.kernel,
        out_type=jax.ShapeDtypeStruct((rows, h), jnp.float32),
        mesh=mesh,
        scratch_types=[
            pltpu.VMEM((2, CHUNK_ROWS, h), jnp.float32),
            pltpu.VMEM((2, CHUNK_ROWS, h), jnp.float32),
            pltpu.VMEM((2, CHUNK_ROWS, h), jnp.float32),
            pltpu.SemaphoreType.DMA((2,)),
            pltpu.SemaphoreType.DMA((2,)),
            pltpu.SemaphoreType.DMA((2,)),
        ],
        compiler_params=pltpu.CompilerParams(use_tc_tiling_on_sc=True),
    )
    def sc_add(x_hbm, t_hbm, o_hbm, xv, tv, ov, semx, semt, semo):
        wid = lax.axis_index("s") * NC + lax.axis_index("c")
        pbase = wid * ppw

        def xrow(m):
            pc = lax.shift_right_logical(m, 2)
            be = lax.bitwise_and(m, b - 1)
            return be * n + pbase + pc * CHUNK_ROWS

        def start_x(m, buf):
            pltpu.async_copy(
                x_hbm.at[pl.ds(xrow(m), CHUNK_ROWS)], xv.at[buf],
                semx.at[buf])

        def wait_x(m, buf):
            pltpu.make_async_copy(
                x_hbm.at[pl.ds(xrow(m), CHUNK_ROWS)], xv.at[buf],
                semx.at[buf]).wait()

        def start_t(pc, tbuf):
            pltpu.async_copy(
                t_hbm.at[pl.ds(pbase + pc * CHUNK_ROWS, CHUNK_ROWS)],
                tv.at[tbuf], semt.at[tbuf])

        def wait_t(pc, tbuf):
            pltpu.make_async_copy(
                t_hbm.at[pl.ds(pbase + pc * CHUNK_ROWS, CHUNK_ROWS)],
                tv.at[tbuf], semt.at[tbuf]).wait()

        def start_out(m, buf):
            pltpu.async_copy(
                ov.at[buf], o_hbm.at[pl.ds(xrow(m), CHUNK_ROWS)],
                semo.at[buf])

        def wait_out(m, buf):
            pltpu.make_async_copy(
                ov.at[buf], o_hbm.at[pl.ds(xrow(m), CHUNK_ROWS)],
                semo.at[buf]).wait()

        start_x(0, 0)
        start_x(1, 1)
        start_t(0, 0)

        def body(m, carry):
            buf = lax.rem(m, 2)
            pc = lax.shift_right_logical(m, 2)
            be = lax.bitwise_and(m, b - 1)
            tbuf = lax.rem(pc, 2)

            wait_x(m, buf)

            @pl.when(be == 0)
            def _():
                wait_t(pc, tbuf)

                @pl.when(pc + 1 < npc)
                def _():
                    start_t(pc + 1, lax.rem(pc + 1, 2))

            @pl.when(m >= 2)
            def _():
                wait_out(m - 2, buf)

            @plsc.parallel_loop(0, CHUNK, step=16, unroll=UNROLL)
            def _add(g):
                r = lax.shift_right_logical(g, 10)
                cc = pl.multiple_of(lax.bitwise_and(g, h - 1), 16)
                ov[buf, r, pl.ds(cc, 16)] = (
                    xv[buf, r, pl.ds(cc, 16)] + tv[tbuf, r, pl.ds(cc, 16)])

            start_out(m, buf)

            @pl.when(m + 2 < nstep)
            def _():
                start_x(m + 2, buf)

            return carry

        lax.fori_loop(0, nstep, body, 0)
        wait_out(nstep - 2, lax.rem(nstep - 2, 2))
        wait_out(nstep - 1, lax.rem(nstep - 1, 2))

    out = sc_add(x2, table)
    return out.reshape(b, n, h)
